# baseline (speedup 1.0000x reference)
import functools

import jax
import jax.numpy as jnp
from jax import lax
from jax.experimental import pallas as pl
from jax.experimental.pallas import tpu as pltpu
from jax.experimental.pallas import tpu_sc as plsc

_B, _C, _H, _W = 4, 96, 384, 384
_N = 8192
_HW = _H * _W
_NC = 2
_NS = 16
_NW = _NC * _NS
_CP = 128
_S_TOTAL = _B * _N
_SPW = _S_TOTAL // _NW
_K = 128
_NCHUNK = _SPW // _K
_G = _K // 16


def _splat16(v):
    return jnp.full((16,), v, jnp.int32)


def _build_sc_call():
    mesh = plsc.VectorSubcoreMesh(core_axis_name="c", subcore_axis_name="s")

    @functools.partial(
        pl.kernel,
        mesh=mesh,
        out_type=jax.ShapeDtypeStruct((_S_TOTAL, _C), jnp.float32),
        scratch_types=[
            pltpu.VMEM((_SPW,), jnp.float32),
            pltpu.VMEM((_SPW,), jnp.float32),
            pltpu.VMEM((8, 16), jnp.float32),
            pltpu.VMEM((4, _K), jnp.int32),
            pltpu.VMEM((4, _K // 8, 128), jnp.float32),
            pltpu.VMEM((4, _K, _CP), jnp.float32),
            pltpu.VMEM((_K, _C), jnp.float32),
            pltpu.SemaphoreType.DMA,
        ],
    )
    def retina(xT_hbm, tx_hbm, ty_hbm, f_hbm, out_hbm,
               tx_v, ty_v, fix_v, idx_v, w_v, rows_v, out_v, sem):
        wid = lax.axis_index("s") * _NC + lax.axis_index("c")
        base_s = wid * _SPW
        b = base_s // _N
        n0 = base_s % _N

        pltpu.sync_copy(tx_hbm.at[pl.ds(n0, _SPW)], tx_v)
        pltpu.sync_copy(ty_hbm.at[pl.ds(n0, _SPW)], ty_v)
        pltpu.sync_copy(f_hbm, fix_v)
        fxv = fix_v[2 * b, :]
        fyv = fix_v[2 * b + 1, :]
        bbase = b * _HW

        def chunk_body(ci, carry):
            s0 = ci * _K
            for g in range(_G):
                src = pl.ds(s0 + g * 16, 16)
                dst = pl.ds(g * 16, 16)
                gx = tx_v[src] + fxv
                gy = ty_v[src] + fyv
                ix = ((gx + 1.0) * _W - 1.0) * 0.5
                iy = ((gy + 1.0) * _H - 1.0) * 0.5
                ti = ix.astype(jnp.int32)
                tf = ti.astype(jnp.float32)
                ix0f = jnp.where(tf > ix, tf - 1.0, tf)
                ti = iy.astype(jnp.int32)
                tf = ti.astype(jnp.float32)
                iy0f = jnp.where(tf > iy, tf - 1.0, tf)
                wx1 = ix - ix0f
                wx0 = 1.0 - wx1
                wy1 = iy - iy0f
                wy0 = 1.0 - wy1
                ix0 = ix0f.astype(jnp.int32)
                ix1 = ix0 + 1
                iy0 = iy0f.astype(jnp.int32)
                iy1 = iy0 + 1
                vx0 = jnp.where((ix0 >= 0) & (ix0 <= _W - 1), 1.0, 0.0)
                vx1 = jnp.where((ix1 >= 0) & (ix1 <= _W - 1), 1.0, 0.0)
                vy0 = jnp.where((iy0 >= 0) & (iy0 <= _H - 1), 1.0, 0.0)
                vy1 = jnp.where((iy1 >= 0) & (iy1 <= _H - 1), 1.0, 0.0)
                cx0 = jnp.clip(ix0, 0, _W - 1)
                cx1 = jnp.clip(ix1, 0, _W - 1)
                cy0 = jnp.clip(iy0, 0, _H - 1)
                cy1 = jnp.clip(iy1, 0, _H - 1)
                idx_v[0, dst] = bbase + cy0 * _W + cx0
                idx_v[1, dst] = bbase + cy0 * _W + cx1
                idx_v[2, dst] = bbase + cy1 * _W + cx0
                idx_v[3, dst] = bbase + cy1 * _W + cx1
                w00 = wx0 * wy0 * vx0 * vy0
                w01 = wx1 * wy0 * vx1 * vy0
                w10 = wx0 * wy1 * vx0 * vy1
                w11 = wx1 * wy1 * vx1 * vy1
                for i in range(16):
                    kr = g * 16 + i
                    row, off = kr // 8, (kr % 8) * 16
                    w_v[0, row, pl.ds(off, 16)] = jnp.full((16,), w00[i], jnp.float32)
                    w_v[1, row, pl.ds(off, 16)] = jnp.full((16,), w01[i], jnp.float32)
                    w_v[2, row, pl.ds(off, 16)] = jnp.full((16,), w10[i], jnp.float32)
                    w_v[3, row, pl.ds(off, 16)] = jnp.full((16,), w11[i], jnp.float32)

            copies = [
                pltpu.async_copy(xT_hbm.at[idx_v.at[j]], rows_v.at[j], sem)
                for j in range(4)
            ]
            for cp in copies:
                cp.wait()

            def blend(k, bcarry):
                krow = k // 8
                koff = pl.multiple_of((k % 8) * 16, 16)
                w0 = w_v[0, krow, pl.ds(koff, 16)]
                w1 = w_v[1, krow, pl.ds(koff, 16)]
                w2 = w_v[2, krow, pl.ds(koff, 16)]
                w3 = w_v[3, krow, pl.ds(koff, 16)]
                for cc in range(_C // 16):
                    csl = pl.ds(cc * 16, 16)
                    acc = w0 * rows_v[0, k, csl]
                    acc = acc + w1 * rows_v[1, k, csl]
                    acc = acc + w2 * rows_v[2, k, csl]
                    acc = acc + w3 * rows_v[3, k, csl]
                    out_v[k, csl] = acc
                return bcarry

            lax.fori_loop(0, _K, blend, 0)
            pltpu.sync_copy(out_v, out_hbm.at[pl.ds(base_s + s0, _K)])
            return carry

        lax.fori_loop(0, _NCHUNK, chunk_body, 0)

    return retina


_sc_retina = _build_sc_call()


def kernel(x, fixations, tess):
    xT = jnp.pad(jnp.transpose(x, (0, 2, 3, 1)).reshape(_B * _HW, _C),
                 ((0, 0), (0, _CP - _C)))
    tx = tess[:, 0] + jnp.zeros((_N,), jnp.float32)
    ty = tess[:, 1] + jnp.zeros((_N,), jnp.float32)
    fpad = jnp.broadcast_to(fixations.reshape(8, 1), (8, 16)) + jnp.zeros(
        (8, 16), jnp.float32)
    out = _sc_retina(xT, tx, ty, fpad)
    return jnp.transpose(out.reshape(_B, _N, _C), (0, 2, 1))
